# baseline (speedup 1.0000x reference)
import functools

import jax
import jax.numpy as jnp
from jax.experimental import pallas as pl
from jax.experimental.pallas import tpu as pltpu

_LANES = 128


def _lenet_kernel(x_ref, w1_ref, b1_ref, w2_ref, b2_ref,
                  wf1_ref, bf1_ref, wf2_ref, bf2_ref, wf3_ref, bf3_ref,
                  o_ref):
    t = x_ref.shape[0]
    xr = x_ref[...]
    x = jnp.concatenate(
        [jnp.transpose(xr[:, c], (1, 0, 2)) for c in range(3)],
        axis=-1)

    s = None
    for di in range(5):
        a = x[di:di + 28].reshape(28 * t, 96)
        m = jnp.dot(a, w1_ref[di], preferred_element_type=jnp.float32)
        s = m if s is None else s + m
    s = s.reshape(14, 2, t, 2 * _LANES)
    p = jnp.maximum(s[:, 0], s[:, 1])
    p = jnp.maximum(p[:, :, :_LANES], p[:, :, _LANES:])
    h1 = jnp.maximum(p + b1_ref[...], 0.0)

    s = None
    for di in range(5):
        a = h1[di:di + 10].reshape(10 * t, _LANES)
        m = jnp.dot(a, w2_ref[di], preferred_element_type=jnp.float32)
        s = m if s is None else s + m
    s = s.reshape(5, 2, t, 2 * _LANES)
    p = jnp.maximum(s[:, 0], s[:, 1])
    p = jnp.maximum(p[:, :, :_LANES], p[:, :, _LANES:])
    h2 = jnp.maximum(p + b2_ref[...], 0.0)

    y = None
    for h in range(5):
        m = jnp.dot(h2[h], wf1_ref[h], preferred_element_type=jnp.float32)
        y = m if y is None else y + m
    y = jnp.maximum(y + bf1_ref[...], 0.0)
    y = jnp.dot(y, wf2_ref[...], preferred_element_type=jnp.float32)
    y = jnp.maximum(y + bf2_ref[...], 0.0)
    y = jnp.dot(y, wf3_ref[...], preferred_element_type=jnp.float32)
    o_ref[...] = y + bf3_ref[...]


def _band_weights(w_ock, c_in, oc, w_in, j_out, rows_out, rows_cw=False):
    w = w_ock[:c_in * 25, :oc].reshape(c_in, 5, 5, oc)
    mats = []
    for di in range(5):
        taps = jnp.transpose(w[:, di], (1, 0, 2))
        halves = []
        for b in (0, 1):
            cols = [jnp.pad(taps, ((2 * j + b, w_in - 5 - 2 * j - b),
                                   (0, 0), (0, 0)))
                    for j in range(j_out)]
            blk = jnp.stack(cols, axis=2).reshape(w_in, c_in, j_out * oc)
            halves.append(jnp.pad(blk, ((0, 0), (0, 0),
                                        (0, _LANES - j_out * oc))))
        m = jnp.concatenate(halves, axis=-1)
        if rows_cw:
            m = jnp.transpose(m, (1, 0, 2))
        mats.append(m.reshape(w_in * c_in, 2 * _LANES))
    wb = jnp.stack(mats)
    pad = rows_out - wb.shape[1]
    if pad:
        wb = jnp.pad(wb, ((0, 0), (0, pad), (0, 0)))
    return wb


@functools.partial(jax.jit, static_argnames=())
def kernel(conv1_w, conv1_b, conv2_w, conv2_b, fc1_w, fc1_b,
           fc2_w, fc2_b, fc3_w, fc3_b, x):
    n = x.shape[0]
    t = 128
    npad = (-n) % t
    xt = x
    if npad:
        xt = jnp.pad(xt, ((0, npad), (0, 0), (0, 0), (0, 0)))
    nblk = (n + npad) // t

    w1 = _band_weights(conv1_w, 3, 6, 32, 14, 96, rows_cw=True)
    w2 = _band_weights(conv2_w, 6, 16, 14, 5, _LANES)
    b1 = jnp.pad(jnp.tile(conv1_b[0, :6], 14), (0, 44)).reshape(1, 1, _LANES)
    b2 = jnp.pad(jnp.tile(conv2_b[0, :16], 5), (0, 48)).reshape(1, 1, _LANES)
    wf1 = jnp.pad(fc1_w.reshape(5, 80, _LANES), ((0, 0), (0, 48), (0, 0)))

    out = pl.pallas_call(
        _lenet_kernel,
        out_shape=jax.ShapeDtypeStruct((n + npad, _LANES), jnp.float32),
        grid=(nblk,),
        in_specs=[
            pl.BlockSpec((t, 3, 32, 32), lambda i: (i, 0, 0, 0)),
            pl.BlockSpec((5, 96, 256), lambda i: (0, 0, 0)),
            pl.BlockSpec((1, 1, _LANES), lambda i: (0, 0, 0)),
            pl.BlockSpec((5, _LANES, 256), lambda i: (0, 0, 0)),
            pl.BlockSpec((1, 1, _LANES), lambda i: (0, 0, 0)),
            pl.BlockSpec((5, _LANES, _LANES), lambda i: (0, 0, 0)),
            pl.BlockSpec((1, _LANES), lambda i: (0, 0)),
            pl.BlockSpec((_LANES, _LANES), lambda i: (0, 0)),
            pl.BlockSpec((1, _LANES), lambda i: (0, 0)),
            pl.BlockSpec((_LANES, _LANES), lambda i: (0, 0)),
            pl.BlockSpec((1, _LANES), lambda i: (0, 0)),
        ],
        out_specs=pl.BlockSpec((t, _LANES), lambda i: (i, 0)),
        compiler_params=pltpu.CompilerParams(
            dimension_semantics=("parallel",)),
    )(xt, w1, b1, w2, b2, wf1, fc1_b, fc2_w, fc2_b, fc3_w, fc3_b)
    return out[:n, :10]


# speedup vs baseline: 1.3099x; 1.3099x over previous
import functools

import jax
import jax.numpy as jnp
from jax.experimental import pallas as pl
from jax.experimental.pallas import tpu as pltpu

_LANES = 128


def _lenet_kernel(x_ref, w1_ref, b1_ref, w2_ref, b2_ref,
                  wf1_ref, bf1_ref, wf2_ref, bf2_ref, wf3_ref, bf3_ref,
                  o_ref):
    t = x_ref.shape[0] // 3
    xr = x_ref[...].reshape(t, 3, 32, 32)
    x = jnp.concatenate(
        [jnp.transpose(xr[:, c], (1, 0, 2)) for c in range(3)],
        axis=-1)

    s = None
    for di in range(5):
        a = x[di:di + 28].reshape(28 * t, 96)
        m = jnp.dot(a, w1_ref[di], preferred_element_type=jnp.float32)
        s = m if s is None else s + m
    s = s.reshape(14, 2, t, 2 * _LANES)
    p = jnp.maximum(s[:, 0], s[:, 1])
    p = jnp.maximum(p[:, :, :_LANES], p[:, :, _LANES:])
    h1 = jnp.maximum(p + b1_ref[...], 0.0)

    s = None
    for di in range(5):
        a = h1[di:di + 10].reshape(10 * t, _LANES)
        m = jnp.dot(a, w2_ref[di], preferred_element_type=jnp.float32)
        s = m if s is None else s + m
    s = s.reshape(5, 2, t, 2 * _LANES)
    p = jnp.maximum(s[:, 0], s[:, 1])
    p = jnp.maximum(p[:, :, :_LANES], p[:, :, _LANES:])
    h2 = jnp.maximum(p + b2_ref[...], 0.0)

    y = None
    for h in range(5):
        m = jnp.dot(h2[h], wf1_ref[h], preferred_element_type=jnp.float32)
        y = m if y is None else y + m
    y = jnp.maximum(y + bf1_ref[...], 0.0)
    y = jnp.dot(y, wf2_ref[...], preferred_element_type=jnp.float32)
    y = jnp.maximum(y + bf2_ref[...], 0.0)
    y = jnp.dot(y, wf3_ref[...], preferred_element_type=jnp.float32)
    o_ref[...] = y + bf3_ref[...]


def _band_weights(w_ock, c_in, oc, w_in, j_out, rows_out, rows_cw=False):
    w = w_ock[:c_in * 25, :oc].reshape(c_in, 5, 5, oc)
    mats = []
    for di in range(5):
        taps = jnp.transpose(w[:, di], (1, 0, 2))
        halves = []
        for b in (0, 1):
            cols = [jnp.pad(taps, ((2 * j + b, w_in - 5 - 2 * j - b),
                                   (0, 0), (0, 0)))
                    for j in range(j_out)]
            blk = jnp.stack(cols, axis=2).reshape(w_in, c_in, j_out * oc)
            halves.append(jnp.pad(blk, ((0, 0), (0, 0),
                                        (0, _LANES - j_out * oc))))
        m = jnp.concatenate(halves, axis=-1)
        if rows_cw:
            m = jnp.transpose(m, (1, 0, 2))
        mats.append(m.reshape(w_in * c_in, 2 * _LANES))
    wb = jnp.stack(mats)
    pad = rows_out - wb.shape[1]
    if pad:
        wb = jnp.pad(wb, ((0, 0), (0, pad), (0, 0)))
    return wb


@functools.partial(jax.jit, static_argnames=())
def kernel(conv1_w, conv1_b, conv2_w, conv2_b, fc1_w, fc1_b,
           fc2_w, fc2_b, fc3_w, fc3_b, x):
    n = x.shape[0]
    t = 128
    npad = (-n) % t
    xt = x
    if npad:
        xt = jnp.pad(xt, ((0, npad), (0, 0), (0, 0), (0, 0)))
    xt = xt.reshape((n + npad) * 3, 32, 32)
    nblk = (n + npad) // t

    w1 = _band_weights(conv1_w, 3, 6, 32, 14, 96, rows_cw=True)
    w2 = _band_weights(conv2_w, 6, 16, 14, 5, _LANES)
    b1 = jnp.pad(jnp.tile(conv1_b[0, :6], 14), (0, 44)).reshape(1, 1, _LANES)
    b2 = jnp.pad(jnp.tile(conv2_b[0, :16], 5), (0, 48)).reshape(1, 1, _LANES)
    wf1 = jnp.pad(fc1_w.reshape(5, 80, _LANES), ((0, 0), (0, 48), (0, 0)))

    out = pl.pallas_call(
        _lenet_kernel,
        out_shape=jax.ShapeDtypeStruct((n + npad, _LANES), jnp.float32),
        grid=(nblk,),
        in_specs=[
            pl.BlockSpec((3 * t, 32, 32), lambda i: (i, 0, 0)),
            pl.BlockSpec((5, 96, 256), lambda i: (0, 0, 0)),
            pl.BlockSpec((1, 1, _LANES), lambda i: (0, 0, 0)),
            pl.BlockSpec((5, _LANES, 256), lambda i: (0, 0, 0)),
            pl.BlockSpec((1, 1, _LANES), lambda i: (0, 0, 0)),
            pl.BlockSpec((5, _LANES, _LANES), lambda i: (0, 0, 0)),
            pl.BlockSpec((1, _LANES), lambda i: (0, 0)),
            pl.BlockSpec((_LANES, _LANES), lambda i: (0, 0)),
            pl.BlockSpec((1, _LANES), lambda i: (0, 0)),
            pl.BlockSpec((_LANES, _LANES), lambda i: (0, 0)),
            pl.BlockSpec((1, _LANES), lambda i: (0, 0)),
        ],
        out_specs=pl.BlockSpec((t, _LANES), lambda i: (i, 0)),
        compiler_params=pltpu.CompilerParams(
            dimension_semantics=("parallel",)),
    )(xt, w1, b1, w2, b2, wf1, fc1_b, fc2_w, fc2_b, fc3_w, fc3_b)
    return out[:n, :10]


# speedup vs baseline: 1.5384x; 1.1745x over previous
import functools

import jax
import jax.numpy as jnp
from jax.experimental import pallas as pl
from jax.experimental.pallas import tpu as pltpu

_LANES = 128


def _lenet_kernel(x_ref, w1_ref, b1_ref, w2_ref, b2_ref,
                  wf1_ref, bf1_ref, wf2_ref, bf2_ref, wf3_ref, bf3_ref,
                  o_ref):
    t = x_ref.shape[0] // 3
    xr = x_ref[...].reshape(t, 3, 32, 32)
    xr = xr.astype(jnp.bfloat16)
    x = jnp.concatenate(
        [jnp.transpose(xr[:, c], (1, 0, 2)) for c in range(3)],
        axis=-1)

    s = None
    for di in range(5):
        a = x[di:di + 28].reshape(28 * t, 96)
        m = jnp.dot(a, w1_ref[di], preferred_element_type=jnp.float32)
        s = m if s is None else s + m
    s = s.reshape(14, 2, t, 2 * _LANES)
    p = jnp.maximum(s[:, 0], s[:, 1])
    p = jnp.maximum(p[:, :, :_LANES], p[:, :, _LANES:])
    h1 = jnp.maximum(p + b1_ref[...], 0.0).astype(jnp.bfloat16)

    s = None
    for di in range(5):
        a = h1[di:di + 10].reshape(10 * t, _LANES)
        m = jnp.dot(a, w2_ref[di], preferred_element_type=jnp.float32)
        s = m if s is None else s + m
    s = s.reshape(5, 2, t, 2 * _LANES)
    p = jnp.maximum(s[:, 0], s[:, 1])
    p = jnp.maximum(p[:, :, :_LANES], p[:, :, _LANES:])
    h2 = jnp.maximum(p + b2_ref[...], 0.0)

    y = None
    for h in range(5):
        m = jnp.dot(h2[h], wf1_ref[h], preferred_element_type=jnp.float32)
        y = m if y is None else y + m
    y = jnp.maximum(y + bf1_ref[...], 0.0)
    y = jnp.dot(y, wf2_ref[...], preferred_element_type=jnp.float32)
    y = jnp.maximum(y + bf2_ref[...], 0.0)
    y = jnp.dot(y, wf3_ref[...], preferred_element_type=jnp.float32)
    o_ref[...] = (y + bf3_ref[...])[:, :10]


def _band_weights(w_ock, c_in, oc, w_in, j_out, rows_out, rows_cw=False):
    w = w_ock[:c_in * 25, :oc].reshape(c_in, 5, 5, oc)
    mats = []
    for di in range(5):
        taps = jnp.transpose(w[:, di], (1, 0, 2))
        halves = []
        for b in (0, 1):
            cols = [jnp.pad(taps, ((2 * j + b, w_in - 5 - 2 * j - b),
                                   (0, 0), (0, 0)))
                    for j in range(j_out)]
            blk = jnp.stack(cols, axis=2).reshape(w_in, c_in, j_out * oc)
            halves.append(jnp.pad(blk, ((0, 0), (0, 0),
                                        (0, _LANES - j_out * oc))))
        m = jnp.concatenate(halves, axis=-1)
        if rows_cw:
            m = jnp.transpose(m, (1, 0, 2))
        mats.append(m.reshape(w_in * c_in, 2 * _LANES))
    wb = jnp.stack(mats)
    pad = rows_out - wb.shape[1]
    if pad:
        wb = jnp.pad(wb, ((0, 0), (0, pad), (0, 0)))
    return wb


@functools.partial(jax.jit, static_argnames=())
def kernel(conv1_w, conv1_b, conv2_w, conv2_b, fc1_w, fc1_b,
           fc2_w, fc2_b, fc3_w, fc3_b, x):
    n = x.shape[0]
    t = 128
    npad = (-n) % t
    xt = x
    if npad:
        xt = jnp.pad(xt, ((0, npad), (0, 0), (0, 0), (0, 0)))
    xt = xt.reshape((n + npad) * 3, 32, 32)
    nblk = (n + npad) // t

    w1 = _band_weights(conv1_w, 3, 6, 32, 14, 96,
                       rows_cw=True).astype(jnp.bfloat16)
    w2 = _band_weights(conv2_w, 6, 16, 14, 5,
                       _LANES).astype(jnp.bfloat16)
    b1 = jnp.pad(jnp.tile(conv1_b[0, :6], 14), (0, 44)).reshape(1, 1, _LANES)
    b2 = jnp.pad(jnp.tile(conv2_b[0, :16], 5), (0, 48)).reshape(1, 1, _LANES)
    wf1 = jnp.pad(fc1_w.reshape(5, 80, _LANES), ((0, 0), (0, 48), (0, 0)))

    out = pl.pallas_call(
        _lenet_kernel,
        out_shape=jax.ShapeDtypeStruct((n + npad, 10), jnp.float32),
        grid=(nblk,),
        in_specs=[
            pl.BlockSpec((3 * t, 32, 32), lambda i: (i, 0, 0)),
            pl.BlockSpec((5, 96, 256), lambda i: (0, 0, 0)),
            pl.BlockSpec((1, 1, _LANES), lambda i: (0, 0, 0)),
            pl.BlockSpec((5, _LANES, 256), lambda i: (0, 0, 0)),
            pl.BlockSpec((1, 1, _LANES), lambda i: (0, 0, 0)),
            pl.BlockSpec((5, _LANES, _LANES), lambda i: (0, 0, 0)),
            pl.BlockSpec((1, _LANES), lambda i: (0, 0)),
            pl.BlockSpec((_LANES, _LANES), lambda i: (0, 0)),
            pl.BlockSpec((1, _LANES), lambda i: (0, 0)),
            pl.BlockSpec((_LANES, _LANES), lambda i: (0, 0)),
            pl.BlockSpec((1, _LANES), lambda i: (0, 0)),
        ],
        out_specs=pl.BlockSpec((t, 10), lambda i: (i, 0)),
        compiler_params=pltpu.CompilerParams(
            dimension_semantics=("parallel",)),
    )(xt, w1, b1, w2, b2, wf1, fc1_b, fc2_w, fc2_b, fc3_w, fc3_b)
    return out[:n]


# speedup vs baseline: 1.9193x; 1.2475x over previous
import functools

import jax
import jax.numpy as jnp
from jax.experimental import pallas as pl
from jax.experimental.pallas import tpu as pltpu

_LANES = 128


def _lenet_kernel(x_ref, w1_ref, b1_ref, w2_ref, b2_ref,
                  wf1_ref, bf1_ref, wf2_ref, bf2_ref, wf3_ref, bf3_ref,
                  o_ref):
    t = x_ref.shape[0] // 3
    xr = x_ref[...].reshape(t, 3, 32, 32)
    xr = xr.astype(jnp.bfloat16)
    x = jnp.concatenate(
        [jnp.transpose(xr[:, c], (1, 0, 2)) for c in range(3)]
        + [jnp.zeros((32, t, 32), jnp.bfloat16)],
        axis=-1)

    a = jnp.concatenate([x[di:di + 28] for di in range(5)], axis=-1)
    s = jnp.dot(a.reshape(28 * t, 5 * _LANES), w1_ref[...],
                preferred_element_type=jnp.float32)
    s = s.reshape(14, 2, t, 2 * _LANES)
    p = jnp.maximum(s[:, 0], s[:, 1])
    p = jnp.maximum(p[:, :, :_LANES], p[:, :, _LANES:])
    h1 = jnp.maximum(p + b1_ref[...], 0.0).astype(jnp.bfloat16)

    a = jnp.concatenate([h1[di:di + 10] for di in range(5)], axis=-1)
    s = jnp.dot(a.reshape(10 * t, 5 * _LANES), w2_ref[...],
                preferred_element_type=jnp.float32)
    s = s.reshape(5, 2, t, 2 * _LANES)
    p = jnp.maximum(s[:, 0], s[:, 1])
    p = jnp.maximum(p[:, :, :_LANES], p[:, :, _LANES:])
    h2 = jnp.maximum(p + b2_ref[...], 0.0)

    a = jnp.concatenate([h2[h] for h in range(5)], axis=-1)
    y = jnp.dot(a, wf1_ref[...], preferred_element_type=jnp.float32)
    y = jnp.maximum(y + bf1_ref[...], 0.0)
    y = jnp.dot(y, wf2_ref[...], preferred_element_type=jnp.float32)
    y = jnp.maximum(y + bf2_ref[...], 0.0)
    y = jnp.dot(y, wf3_ref[...], preferred_element_type=jnp.float32)
    o_ref[...] = (y + bf3_ref[...])[:, :10]


def _band_weights(w_ock, c_in, oc, w_in, j_out, rows_out, rows_cw=False):
    w = w_ock[:c_in * 25, :oc].reshape(c_in, 5, 5, oc)
    mats = []
    for di in range(5):
        taps = jnp.transpose(w[:, di], (1, 0, 2))
        halves = []
        for b in (0, 1):
            cols = [jnp.pad(taps, ((2 * j + b, w_in - 5 - 2 * j - b),
                                   (0, 0), (0, 0)))
                    for j in range(j_out)]
            blk = jnp.stack(cols, axis=2).reshape(w_in, c_in, j_out * oc)
            halves.append(jnp.pad(blk, ((0, 0), (0, 0),
                                        (0, _LANES - j_out * oc))))
        m = jnp.concatenate(halves, axis=-1)
        if rows_cw:
            m = jnp.transpose(m, (1, 0, 2))
        mats.append(m.reshape(w_in * c_in, 2 * _LANES))
    wb = jnp.stack(mats)
    pad = rows_out - wb.shape[1]
    if pad:
        wb = jnp.pad(wb, ((0, 0), (0, pad), (0, 0)))
    return wb


@functools.partial(jax.jit, static_argnames=())
def kernel(conv1_w, conv1_b, conv2_w, conv2_b, fc1_w, fc1_b,
           fc2_w, fc2_b, fc3_w, fc3_b, x):
    n = x.shape[0]
    t = 128
    npad = (-n) % t
    xt = x
    if npad:
        xt = jnp.pad(xt, ((0, npad), (0, 0), (0, 0), (0, 0)))
    xt = xt.reshape((n + npad) * 3, 32, 32)
    nblk = (n + npad) // t

    w1 = _band_weights(conv1_w, 3, 6, 32, 14, _LANES,
                       rows_cw=True).astype(jnp.bfloat16).reshape(640, 256)
    w2 = _band_weights(conv2_w, 6, 16, 14, 5,
                       _LANES).astype(jnp.bfloat16).reshape(640, 256)
    b1 = jnp.pad(jnp.tile(conv1_b[0, :6], 14), (0, 44)).reshape(1, 1, _LANES)
    b2 = jnp.pad(jnp.tile(conv2_b[0, :16], 5), (0, 48)).reshape(1, 1, _LANES)
    wf1 = jnp.pad(fc1_w.reshape(5, 80, _LANES),
                  ((0, 0), (0, 48), (0, 0))).reshape(640, _LANES)

    out = pl.pallas_call(
        _lenet_kernel,
        out_shape=jax.ShapeDtypeStruct((n + npad, 10), jnp.float32),
        grid=(nblk,),
        in_specs=[
            pl.BlockSpec((3 * t, 32, 32), lambda i: (i, 0, 0)),
            pl.BlockSpec((640, 256), lambda i: (0, 0)),
            pl.BlockSpec((1, 1, _LANES), lambda i: (0, 0, 0)),
            pl.BlockSpec((640, 256), lambda i: (0, 0)),
            pl.BlockSpec((1, 1, _LANES), lambda i: (0, 0, 0)),
            pl.BlockSpec((640, _LANES), lambda i: (0, 0)),
            pl.BlockSpec((1, _LANES), lambda i: (0, 0)),
            pl.BlockSpec((_LANES, _LANES), lambda i: (0, 0)),
            pl.BlockSpec((1, _LANES), lambda i: (0, 0)),
            pl.BlockSpec((_LANES, _LANES), lambda i: (0, 0)),
            pl.BlockSpec((1, _LANES), lambda i: (0, 0)),
        ],
        out_specs=pl.BlockSpec((t, 10), lambda i: (i, 0)),
        compiler_params=pltpu.CompilerParams(
            dimension_semantics=("parallel",)),
    )(xt, w1, b1, w2, b2, wf1, fc1_b, fc2_w, fc2_b, fc3_w, fc3_b)
    return out[:n]


# speedup vs baseline: 1.9217x; 1.0013x over previous
import functools

import jax
import jax.numpy as jnp
import numpy as np
from jax.experimental import pallas as pl
from jax.experimental.pallas import tpu as pltpu

_LANES = 128


def _band_selector(w_in, j_out):
    dj, w, b, j = np.ogrid[0:5, 0:w_in, 0:2, 0:j_out]
    return (w == 2 * j + b + dj).astype(np.float32)


def _lenet_kernel(x_ref, w1_ref, b1_ref, w2_ref, b2_ref,
                  wf1_ref, bf1_ref, wf2_ref, bf2_ref, wf3_ref, bf3_ref,
                  o_ref):
    t = x_ref.shape[0] // 3
    xr = x_ref[...].reshape(t, 3, 32, 32)
    xr = xr.astype(jnp.bfloat16)
    x = jnp.concatenate(
        [jnp.transpose(xr[:, c], (1, 0, 2)) for c in range(3)]
        + [jnp.zeros((32, t, 32), jnp.bfloat16)],
        axis=-1)

    a = jnp.concatenate([x[di:di + 28] for di in range(5)], axis=-1)
    s = jnp.dot(a.reshape(28 * t, 5 * _LANES), w1_ref[...],
                preferred_element_type=jnp.float32)
    s = s.reshape(14, 2, t, 2 * _LANES)
    p = jnp.maximum(s[:, 0], s[:, 1])
    p = jnp.maximum(p[:, :, :_LANES], p[:, :, _LANES:])
    h1 = jnp.maximum(p + b1_ref[...], 0.0).astype(jnp.bfloat16)

    a = jnp.concatenate([h1[di:di + 10] for di in range(5)], axis=-1)
    s = jnp.dot(a.reshape(10 * t, 5 * _LANES), w2_ref[...],
                preferred_element_type=jnp.float32)
    s = s.reshape(5, 2, t, 2 * _LANES)
    p = jnp.maximum(s[:, 0], s[:, 1])
    p = jnp.maximum(p[:, :, :_LANES], p[:, :, _LANES:])
    h2 = jnp.maximum(p + b2_ref[...], 0.0)

    a = jnp.concatenate([h2[h] for h in range(5)], axis=-1)
    y = jnp.dot(a, wf1_ref[...], preferred_element_type=jnp.float32)
    y = jnp.maximum(y + bf1_ref[...], 0.0)
    y = jnp.dot(y, wf2_ref[...], preferred_element_type=jnp.float32)
    y = jnp.maximum(y + bf2_ref[...], 0.0)
    y = jnp.dot(y, wf3_ref[...], preferred_element_type=jnp.float32)
    o_ref[...] = (y + bf3_ref[...])[:, :10]


def _band_weights(w_ock, c_in, oc, w_in, j_out, rows_cw=False):
    wt = w_ock[:c_in * 25, :oc].reshape(c_in, 5, 5, oc)
    wt = jnp.transpose(wt, (2, 1, 0, 3))
    s = jnp.asarray(_band_selector(w_in, j_out))
    if rows_cw:
        wb = jnp.einsum('dwbj,dico->icwbjo', s, wt)
        rows = c_in * w_in
    else:
        wb = jnp.einsum('dwbj,dico->iwcbjo', s, wt)
        rows = w_in * c_in
    wb = wb.reshape(5, rows, 2, j_out * oc)
    wb = jnp.pad(wb, ((0, 0), (0, _LANES - rows), (0, 0),
                      (0, _LANES - j_out * oc)))
    return wb.reshape(5 * _LANES, 2 * _LANES)


@functools.partial(jax.jit, static_argnames=())
def kernel(conv1_w, conv1_b, conv2_w, conv2_b, fc1_w, fc1_b,
           fc2_w, fc2_b, fc3_w, fc3_b, x):
    n = x.shape[0]
    t = 128
    npad = (-n) % t
    xt = x
    if npad:
        xt = jnp.pad(xt, ((0, npad), (0, 0), (0, 0), (0, 0)))
    xt = xt.reshape((n + npad) * 3, 32, 32)
    nblk = (n + npad) // t

    w1 = _band_weights(conv1_w, 3, 6, 32, 14,
                       rows_cw=True).astype(jnp.bfloat16)
    w2 = _band_weights(conv2_w, 6, 16, 14, 5).astype(jnp.bfloat16)
    b1 = jnp.pad(jnp.tile(conv1_b[0, :6], 14), (0, 44)).reshape(1, 1, _LANES)
    b2 = jnp.pad(jnp.tile(conv2_b[0, :16], 5), (0, 48)).reshape(1, 1, _LANES)
    wf1 = jnp.pad(fc1_w.reshape(5, 80, _LANES),
                  ((0, 0), (0, 48), (0, 0))).reshape(640, _LANES)

    out = pl.pallas_call(
        _lenet_kernel,
        out_shape=jax.ShapeDtypeStruct((n + npad, 10), jnp.float32),
        grid=(nblk,),
        in_specs=[
            pl.BlockSpec((3 * t, 32, 32), lambda i: (i, 0, 0)),
            pl.BlockSpec((640, 256), lambda i: (0, 0)),
            pl.BlockSpec((1, 1, _LANES), lambda i: (0, 0, 0)),
            pl.BlockSpec((640, 256), lambda i: (0, 0)),
            pl.BlockSpec((1, 1, _LANES), lambda i: (0, 0, 0)),
            pl.BlockSpec((640, _LANES), lambda i: (0, 0)),
            pl.BlockSpec((1, _LANES), lambda i: (0, 0)),
            pl.BlockSpec((_LANES, _LANES), lambda i: (0, 0)),
            pl.BlockSpec((1, _LANES), lambda i: (0, 0)),
            pl.BlockSpec((_LANES, _LANES), lambda i: (0, 0)),
            pl.BlockSpec((1, _LANES), lambda i: (0, 0)),
        ],
        out_specs=pl.BlockSpec((t, 10), lambda i: (i, 0)),
        compiler_params=pltpu.CompilerParams(
            dimension_semantics=("parallel",)),
    )(xt, w1, b1, w2, b2, wf1, fc1_b, fc2_w, fc2_b, fc3_w, fc3_b)
    return out[:n]


# speedup vs baseline: 1.9826x; 1.0317x over previous
import functools

import jax
import jax.numpy as jnp
import numpy as np
from jax.experimental import pallas as pl
from jax.experimental.pallas import tpu as pltpu

_LANES = 128


def _band_selector(w_in, j_out):
    dj, w, b, j = np.ogrid[0:5, 0:w_in, 0:2, 0:j_out]
    return (w == 2 * j + b + dj).astype(np.float32)


def _lenet_kernel(x_ref, w1_ref, b1_ref, w2_ref, b2_ref,
                  wf1_ref, bf1_ref, wf2_ref, bf2_ref, wf3_ref, bf3_ref,
                  o_ref):
    t = x_ref.shape[0] // 3
    xr = x_ref[...].reshape(t, 3, 32, 32)
    xr = xr.astype(jnp.bfloat16)
    x = jnp.concatenate(
        [jnp.transpose(xr[:, c], (1, 0, 2)) for c in range(3)]
        + [jnp.zeros((32, t, 32), jnp.bfloat16)],
        axis=-1)

    a = jnp.concatenate([x[di:di + 28] for di in range(5)], axis=-1)
    s = jnp.dot(a.reshape(28 * t, 5 * _LANES), w1_ref[...],
                preferred_element_type=jnp.float32)
    s = s.reshape(14, 2, t, 2 * _LANES)
    p = jnp.maximum(s[:, 0], s[:, 1])
    p = jnp.maximum(p[:, :, :_LANES], p[:, :, _LANES:])
    h1 = jnp.maximum(p + b1_ref[...], 0.0).astype(jnp.bfloat16)

    a = jnp.concatenate([h1[di:di + 10] for di in range(5)], axis=-1)
    s = jnp.dot(a.reshape(10 * t, 5 * _LANES), w2_ref[...],
                preferred_element_type=jnp.float32)
    s = s.reshape(5, 2, t, 2 * _LANES)
    p = jnp.maximum(s[:, 0], s[:, 1])
    p = jnp.maximum(p[:, :, :_LANES], p[:, :, _LANES:])
    h2 = jnp.maximum(p + b2_ref[...], 0.0)

    a = jnp.concatenate([h2[h] for h in range(5)], axis=-1)
    y = jnp.dot(a, wf1_ref[...], preferred_element_type=jnp.float32)
    y = jnp.maximum(y + bf1_ref[...], 0.0)
    y = jnp.dot(y, wf2_ref[...], preferred_element_type=jnp.float32)
    y = jnp.maximum(y + bf2_ref[...], 0.0)
    y = jnp.dot(y, wf3_ref[...], preferred_element_type=jnp.float32)
    o_ref[...] = (y + bf3_ref[...])[:, :10]


def _band_weights(w_ock, c_in, oc, w_in, j_out, rows_cw=False):
    wt = w_ock[:c_in * 25, :oc].reshape(c_in, 5, 5, oc)
    wt = jnp.transpose(wt, (2, 1, 0, 3))
    s = jnp.asarray(_band_selector(w_in, j_out))
    if rows_cw:
        wb = jnp.einsum('dwbj,dico->icwbjo', s, wt)
        rows = c_in * w_in
    else:
        wb = jnp.einsum('dwbj,dico->iwcbjo', s, wt)
        rows = w_in * c_in
    wb = wb.reshape(5, rows, 2, j_out * oc)
    wb = jnp.pad(wb, ((0, 0), (0, _LANES - rows), (0, 0),
                      (0, _LANES - j_out * oc)))
    return wb.reshape(5 * _LANES, 2 * _LANES)


@functools.partial(jax.jit, static_argnames=())
def kernel(conv1_w, conv1_b, conv2_w, conv2_b, fc1_w, fc1_b,
           fc2_w, fc2_b, fc3_w, fc3_b, x):
    n = x.shape[0]
    t = 256 if n % 256 == 0 else 128
    npad = (-n) % t
    xt = x
    if npad:
        xt = jnp.pad(xt, ((0, npad), (0, 0), (0, 0), (0, 0)))
    xt = xt.reshape((n + npad) * 3, 32, 32)
    nblk = (n + npad) // t

    w1 = _band_weights(conv1_w, 3, 6, 32, 14,
                       rows_cw=True).astype(jnp.bfloat16)
    w2 = _band_weights(conv2_w, 6, 16, 14, 5).astype(jnp.bfloat16)
    b1 = jnp.pad(jnp.tile(conv1_b[0, :6], 14), (0, 44)).reshape(1, 1, _LANES)
    b2 = jnp.pad(jnp.tile(conv2_b[0, :16], 5), (0, 48)).reshape(1, 1, _LANES)
    wf1 = jnp.pad(fc1_w.reshape(5, 80, _LANES),
                  ((0, 0), (0, 48), (0, 0))).reshape(640, _LANES)

    out = pl.pallas_call(
        _lenet_kernel,
        out_shape=jax.ShapeDtypeStruct((n + npad, 10), jnp.float32),
        grid=(nblk,),
        in_specs=[
            pl.BlockSpec((3 * t, 32, 32), lambda i: (i, 0, 0)),
            pl.BlockSpec((640, 256), lambda i: (0, 0)),
            pl.BlockSpec((1, 1, _LANES), lambda i: (0, 0, 0)),
            pl.BlockSpec((640, 256), lambda i: (0, 0)),
            pl.BlockSpec((1, 1, _LANES), lambda i: (0, 0, 0)),
            pl.BlockSpec((640, _LANES), lambda i: (0, 0)),
            pl.BlockSpec((1, _LANES), lambda i: (0, 0)),
            pl.BlockSpec((_LANES, _LANES), lambda i: (0, 0)),
            pl.BlockSpec((1, _LANES), lambda i: (0, 0)),
            pl.BlockSpec((_LANES, _LANES), lambda i: (0, 0)),
            pl.BlockSpec((1, _LANES), lambda i: (0, 0)),
        ],
        out_specs=pl.BlockSpec((t, 10), lambda i: (i, 0)),
        compiler_params=pltpu.CompilerParams(
            dimension_semantics=("parallel",),
            vmem_limit_bytes=56 * 1024 * 1024),
    )(xt, w1, b1, w2, b2, wf1, fc1_b, fc2_w, fc2_b, fc3_w, fc3_b)
    return out[:n]
